# baseline (speedup 1.0000x reference)
import functools

import jax
import jax.numpy as jnp
from jax import lax
from jax.experimental import pallas as pl
from jax.experimental.pallas import tpu as pltpu
from jax.experimental.pallas import tpu_sc as plsc

_TOPK = 8
_E = 64
_D = 4096
_NC = 2
_NS = 16
_CHUNK = 256


def _matmul_block(w_ref, x_ref, out_ref):
    out_ref[...] = lax.dot_general(
        w_ref[...], x_ref[...], (((1,), (1,)), ((), ())),
        preferred_element_type=jnp.float32)


def _logits_t_tc(x, w, seg, nseg):
    t = x.shape[0]
    ts = t // nseg
    bt = 1024
    nblk = ts // bt
    return pl.pallas_call(
        _matmul_block,
        grid=(nblk,),
        in_specs=[
            pl.BlockSpec((_E, _D), lambda i: (0, 0)),
            pl.BlockSpec((bt, _D), lambda i, s=seg, n=nblk: (s * n + i, 0)),
        ],
        out_specs=pl.BlockSpec((_E, bt), lambda i: (0, i)),
        out_shape=jax.ShapeDtypeStruct((_E, ts), jnp.float32),
    )(w, x)


def _router_sc(logits_t, t):
    nw = _NC * _NS
    tw = t // nw
    c = min(_CHUNK, tw)
    nchunks = tw // c
    mesh = plsc.VectorSubcoreMesh(core_axis_name="c", subcore_axis_name="s",
                                  num_cores=_NC, num_subcores=_NS)

    @functools.partial(
        pl.kernel,
        out_type=(
            jax.ShapeDtypeStruct((t * _TOPK,), jnp.int32),
            jax.ShapeDtypeStruct((t * _TOPK,), jnp.float32),
            jax.ShapeDtypeStruct((t * _E,), jnp.float32),
        ),
        mesh=mesh,
        compiler_params=pltpu.CompilerParams(needs_layout_passes=False),
        scratch_types=[
            pltpu.VMEM((_E, c), jnp.float32),
            pltpu.VMEM((c * _TOPK,), jnp.int32),
            pltpu.VMEM((c * _TOPK,), jnp.float32),
            pltpu.VMEM((c * _E,), jnp.float32),
            pltpu.SemaphoreType.DMA,
        ],
    )
    def run(lg_hbm, idx_hbm, tkw_hbm, full_hbm, lg_v, idx_v, tkw_v, full_v,
            sem):
        wid = lax.axis_index("s") * _NC + lax.axis_index("c")
        lanes = lax.iota(jnp.int32, 16)
        zeros16 = jnp.zeros((16,), jnp.float32)
        neg = jnp.full((16,), -jnp.inf, jnp.float32)

        def chunk_body(ci, carry):
            base = wid * tw + ci * c
            pltpu.async_copy(lg_hbm.at[:, pl.ds(base, c)], lg_v, sem).wait()

            def group_body(g, gc):
                tok = g * 16
                b64 = (tok + lanes) * _E
                b8 = (tok + lanes) * _TOPK
                for j in range(16 * _E // 16):
                    full_v[pl.ds(g * (16 * _E) + j * 16, 16)] = zeros16
                ms, mis = [], []
                for _ in range(_TOPK):
                    pairs = [
                        (lg_v[e, pl.ds(tok, 16)],
                         jnp.full((16,), e, jnp.int32))
                        for e in range(_E)
                    ]
                    while len(pairs) > 1:
                        nxt = []
                        for a in range(0, len(pairs), 2):
                            (va, ia), (vb, ib) = pairs[a], pairs[a + 1]
                            take = vb > va
                            nxt.append((jnp.where(take, vb, va),
                                        jnp.where(take, ib, ia)))
                        pairs = nxt
                    m, mi = pairs[0]
                    plsc.store_scatter(lg_v, [mi, tok + lanes], neg)
                    ms.append(m)
                    mis.append(mi)
                exps = [jnp.exp(mm - ms[0]) for mm in ms]
                tot = exps[0]
                for ex in exps[1:]:
                    tot = tot + ex
                inv = 1.0 / tot
                for kk in range(_TOPK):
                    w = exps[kk] * inv
                    plsc.store_scatter(idx_v, [b8 + kk], mis[kk])
                    plsc.store_scatter(tkw_v, [b8 + kk], w)
                    plsc.store_scatter(full_v, [b64 + mis[kk]], w)
                return gc

            lax.fori_loop(0, c // 16, group_body, 0)
            pltpu.sync_copy(idx_v, idx_hbm.at[pl.ds(base * _TOPK, c * _TOPK)])
            pltpu.sync_copy(tkw_v, tkw_hbm.at[pl.ds(base * _TOPK, c * _TOPK)])
            pltpu.sync_copy(full_v, full_hbm.at[pl.ds(base * _E, c * _E)])
            return carry

        lax.fori_loop(0, nchunks, chunk_body, 0)

    return run(logits_t)


def kernel(x, W):
    t = x.shape[0]
    nseg = 8
    ts = t // nseg
    parts = []
    for s in range(nseg):
        logits_t = _logits_t_tc(x, W, s, nseg)
        idx_f, tkw_f, full_f = _router_sc(logits_t, ts)
        parts.append((idx_f.reshape(ts, _TOPK), tkw_f.reshape(ts, _TOPK),
                      full_f.reshape(ts, _E)))
    return tuple(jnp.concatenate([p[i] for p in parts], axis=0)
                 for i in range(3))


# speedup vs baseline: 1.0034x; 1.0034x over previous
import functools

import jax
import jax.numpy as jnp
from jax import lax
from jax.experimental import pallas as pl
from jax.experimental.pallas import tpu as pltpu
from jax.experimental.pallas import tpu_sc as plsc

_TOPK = 8
_E = 64
_D = 4096
_NC = 2
_NS = 16
_CHUNK = 256


def _matmul_block(w_ref, x_ref, out_ref):
    out_ref[...] = lax.dot_general(
        w_ref[...], x_ref[...], (((1,), (1,)), ((), ())),
        preferred_element_type=jnp.float32)


def _logits_t_tc(x, w, seg, nseg):
    t = x.shape[0]
    ts = t // nseg
    bt = 1024
    nblk = ts // bt
    return pl.pallas_call(
        _matmul_block,
        grid=(nblk,),
        in_specs=[
            pl.BlockSpec((_E, _D), lambda i: (0, 0)),
            pl.BlockSpec((bt, _D), lambda i, s=seg, n=nblk: (s * n + i, 0)),
        ],
        out_specs=pl.BlockSpec((_E, bt), lambda i: (0, i)),
        out_shape=jax.ShapeDtypeStruct((_E, ts), jnp.float32),
    )(w, x)


def _router_sc(logits_t, t):
    nw = _NC * _NS
    tw = t // nw
    c = min(_CHUNK, tw)
    nchunks = tw // c
    mesh = plsc.VectorSubcoreMesh(core_axis_name="c", subcore_axis_name="s",
                                  num_cores=_NC, num_subcores=_NS)

    @functools.partial(
        pl.kernel,
        out_type=(
            jax.ShapeDtypeStruct((t * _TOPK,), jnp.int32),
            jax.ShapeDtypeStruct((t * _TOPK,), jnp.float32),
            jax.ShapeDtypeStruct((t * _E,), jnp.float32),
        ),
        mesh=mesh,
        compiler_params=pltpu.CompilerParams(needs_layout_passes=False),
        scratch_types=[
            pltpu.VMEM((_E, c), jnp.float32),
            pltpu.VMEM((c * _TOPK,), jnp.int32),
            pltpu.VMEM((c * _TOPK,), jnp.float32),
            pltpu.VMEM((c * _E,), jnp.float32),
            pltpu.SemaphoreType.DMA,
        ],
    )
    def run(lg_hbm, idx_hbm, tkw_hbm, full_hbm, lg_v, idx_v, tkw_v, full_v,
            sem):
        wid = lax.axis_index("s") * _NC + lax.axis_index("c")
        lanes = lax.iota(jnp.int32, 16)
        zeros16 = jnp.zeros((16,), jnp.float32)
        neg = jnp.full((16,), -jnp.inf, jnp.float32)

        def chunk_body(ci, carry):
            base = wid * tw + ci * c
            pltpu.async_copy(lg_hbm.at[:, pl.ds(base, c)], lg_v, sem).wait()

            def group_body(g, gc):
                tok = g * 16
                b64 = (tok + lanes) * _E
                b8 = (tok + lanes) * _TOPK
                for j in range(16 * _E // 16):
                    full_v[pl.ds(g * (16 * _E) + j * 16, 16)] = zeros16
                ms, mis = [], []
                for _ in range(_TOPK):
                    pairs = [
                        (lg_v[e, pl.ds(tok, 16)],
                         jnp.full((16,), e, jnp.int32))
                        for e in range(_E)
                    ]
                    while len(pairs) > 1:
                        nxt = []
                        for a in range(0, len(pairs), 2):
                            (va, ia), (vb, ib) = pairs[a], pairs[a + 1]
                            take = vb > va
                            nxt.append((jnp.where(take, vb, va),
                                        jnp.where(take, ib, ia)))
                        pairs = nxt
                    m, mi = pairs[0]
                    plsc.store_scatter(lg_v, [mi, tok + lanes], neg)
                    ms.append(m)
                    mis.append(mi)
                exps = [jnp.exp(mm - ms[0]) for mm in ms]
                tot = exps[0]
                for ex in exps[1:]:
                    tot = tot + ex
                inv = 1.0 / tot
                for kk in range(_TOPK):
                    w = exps[kk] * inv
                    plsc.store_scatter(idx_v, [b8 + kk], mis[kk])
                    plsc.store_scatter(tkw_v, [b8 + kk], w)
                    plsc.store_scatter(full_v, [b64 + mis[kk]], w)
                return gc

            lax.fori_loop(0, c // 16, group_body, 0)
            pltpu.sync_copy(idx_v, idx_hbm.at[pl.ds(base * _TOPK, c * _TOPK)])
            pltpu.sync_copy(tkw_v, tkw_hbm.at[pl.ds(base * _TOPK, c * _TOPK)])
            pltpu.sync_copy(full_v, full_hbm.at[pl.ds(base * _E, c * _E)])
            return carry

        lax.fori_loop(0, nchunks, chunk_body, 0)

    return run(logits_t)


def kernel(x, W):
    t = x.shape[0]
    nseg = 2
    ts = t // nseg
    parts = []
    for s in range(nseg):
        logits_t = _logits_t_tc(x, W, s, nseg)
        idx_f, tkw_f, full_f = _router_sc(logits_t, ts)
        parts.append((idx_f.reshape(ts, _TOPK), tkw_f.reshape(ts, _TOPK),
                      full_f.reshape(ts, _E)))
    return tuple(jnp.concatenate([p[i] for p in parts], axis=0)
                 for i in range(3))


# speedup vs baseline: 1.0116x; 1.0081x over previous
import functools

import jax
import jax.numpy as jnp
from jax import lax
from jax.experimental import pallas as pl
from jax.experimental.pallas import tpu as pltpu
from jax.experimental.pallas import tpu_sc as plsc

_TOPK = 8
_E = 64
_D = 4096
_NC = 2
_NS = 16
_CHUNK = 256


def _matmul_block(w_ref, x_ref, out_ref):
    out_ref[...] = lax.dot_general(
        w_ref[...], x_ref[...], (((1,), (1,)), ((), ())),
        preferred_element_type=jnp.float32)


def _logits_t_tc(x, w, tok0, ts):
    bt = 1024
    nblk = ts // bt
    blk0 = tok0 // bt
    return pl.pallas_call(
        _matmul_block,
        grid=(nblk,),
        in_specs=[
            pl.BlockSpec((_E, _D), lambda i: (0, 0)),
            pl.BlockSpec((bt, _D), lambda i, b=blk0: (b + i, 0)),
        ],
        out_specs=pl.BlockSpec((_E, bt), lambda i: (0, i)),
        out_shape=jax.ShapeDtypeStruct((_E, ts), jnp.float32),
    )(w, x)


def _router_sc(logits_t, t):
    nw = _NC * _NS
    tw = t // nw
    c = min(_CHUNK, tw)
    nchunks = tw // c
    mesh = plsc.VectorSubcoreMesh(core_axis_name="c", subcore_axis_name="s",
                                  num_cores=_NC, num_subcores=_NS)

    @functools.partial(
        pl.kernel,
        out_type=(
            jax.ShapeDtypeStruct((t * _TOPK,), jnp.int32),
            jax.ShapeDtypeStruct((t * _TOPK,), jnp.float32),
            jax.ShapeDtypeStruct((t * _E,), jnp.float32),
        ),
        mesh=mesh,
        compiler_params=pltpu.CompilerParams(needs_layout_passes=False),
        scratch_types=[
            pltpu.VMEM((_E, c), jnp.float32),
            pltpu.VMEM((c * _TOPK,), jnp.int32),
            pltpu.VMEM((c * _TOPK,), jnp.float32),
            pltpu.VMEM((c * _E,), jnp.float32),
            pltpu.SemaphoreType.DMA,
        ],
    )
    def run(lg_hbm, idx_hbm, tkw_hbm, full_hbm, lg_v, idx_v, tkw_v, full_v,
            sem):
        wid = lax.axis_index("s") * _NC + lax.axis_index("c")
        lanes = lax.iota(jnp.int32, 16)
        zeros16 = jnp.zeros((16,), jnp.float32)
        neg = jnp.full((16,), -jnp.inf, jnp.float32)

        def chunk_body(ci, carry):
            base = wid * tw + ci * c
            pltpu.async_copy(lg_hbm.at[:, pl.ds(base, c)], lg_v, sem).wait()

            def group_body(g, gc):
                tok = g * 16
                b64 = (tok + lanes) * _E
                b8 = (tok + lanes) * _TOPK
                for j in range(16 * _E // 16):
                    full_v[pl.ds(g * (16 * _E) + j * 16, 16)] = zeros16
                ms, mis = [], []
                for _ in range(_TOPK):
                    pairs = [
                        (lg_v[e, pl.ds(tok, 16)],
                         jnp.full((16,), e, jnp.int32))
                        for e in range(_E)
                    ]
                    while len(pairs) > 1:
                        nxt = []
                        for a in range(0, len(pairs), 2):
                            (va, ia), (vb, ib) = pairs[a], pairs[a + 1]
                            take = vb > va
                            nxt.append((jnp.where(take, vb, va),
                                        jnp.where(take, ib, ia)))
                        pairs = nxt
                    m, mi = pairs[0]
                    plsc.store_scatter(lg_v, [mi, tok + lanes], neg)
                    ms.append(m)
                    mis.append(mi)
                exps = [jnp.exp(mm - ms[0]) for mm in ms]
                tot = exps[0]
                for ex in exps[1:]:
                    tot = tot + ex
                inv = 1.0 / tot
                for kk in range(_TOPK):
                    w = exps[kk] * inv
                    plsc.store_scatter(idx_v, [b8 + kk], mis[kk])
                    plsc.store_scatter(tkw_v, [b8 + kk], w)
                    plsc.store_scatter(full_v, [b64 + mis[kk]], w)
                return gc

            lax.fori_loop(0, c // 16, group_body, 0)
            pltpu.sync_copy(idx_v, idx_hbm.at[pl.ds(base * _TOPK, c * _TOPK)])
            pltpu.sync_copy(tkw_v, tkw_hbm.at[pl.ds(base * _TOPK, c * _TOPK)])
            pltpu.sync_copy(full_v, full_hbm.at[pl.ds(base * _E, c * _E)])
            return carry

        lax.fori_loop(0, nchunks, chunk_body, 0)

    return run(logits_t)


def kernel(x, W):
    t = x.shape[0]
    sizes = (16384, 8192, 4096, 4096)
    assert sum(sizes) == t
    parts = []
    tok0 = 0
    for ts in sizes:
        logits_t = _logits_t_tc(x, W, tok0, ts)
        idx_f, tkw_f, full_f = _router_sc(logits_t, ts)
        parts.append((idx_f.reshape(ts, _TOPK), tkw_f.reshape(ts, _TOPK),
                      full_f.reshape(ts, _E)))
        tok0 += ts
    return tuple(jnp.concatenate([p[i] for p in parts], axis=0)
                 for i in range(3))
